# baseline (speedup 1.0000x reference)
import functools

import jax
import jax.numpy as jnp
from jax import lax
from jax.experimental import pallas as pl
from jax.experimental.pallas import tpu as pltpu
from jax.experimental.pallas import tpu_sc as plsc

TILE_M = 512
N_CORES = 2
N_SUBCORES = 16
N_WORKERS = N_CORES * N_SUBCORES


def _argmax_kernel(x_ref, wt_ref, b_ref, idx_ref):
    logits = jnp.dot(x_ref[...], wt_ref[...],
                     preferred_element_type=jnp.float32) + b_ref[...]
    m = jnp.max(logits, axis=1, keepdims=True)
    e = logits.shape[1]
    iota = jax.lax.broadcasted_iota(jnp.int32, logits.shape, 1)
    idx_ref[...] = jnp.min(jnp.where(logits == m, iota, e), axis=1,
                           keepdims=True)


def _tc_argmax(x, W, b):
    tokens, d_model = x.shape
    n_experts = W.shape[0]
    grid = (tokens // TILE_M,)
    return pl.pallas_call(
        _argmax_kernel,
        grid=grid,
        in_specs=[
            pl.BlockSpec((TILE_M, d_model), lambda i: (i, 0)),
            pl.BlockSpec((d_model, n_experts), lambda i: (0, 0)),
            pl.BlockSpec((1, n_experts), lambda i: (0, 0)),
        ],
        out_specs=pl.BlockSpec((TILE_M, 1), lambda i: (i, 0)),
        out_shape=jax.ShapeDtypeStruct((tokens, 1), jnp.int32),
        compiler_params=pltpu.CompilerParams(
            dimension_semantics=("arbitrary",),
        ),
    )(x, W.T, b.reshape(1, n_experts))


def _sc_onehot(idx_flat, tokens, n_experts):
    rpw = tokens // N_WORKERS
    chunk = rpw * n_experts
    mesh = plsc.VectorSubcoreMesh(core_axis_name="c", subcore_axis_name="s")

    @functools.partial(
        pl.kernel, mesh=mesh,
        out_type=jax.ShapeDtypeStruct((tokens * n_experts,), jnp.float32),
        scratch_types=[
            pltpu.VMEM((rpw,), jnp.int32),
            pltpu.VMEM((chunk,), jnp.float32),
            pltpu.VMEM((2, 128), jnp.int32),
            pltpu.VMEM((128,), jnp.float32),
            pltpu.SemaphoreType.DMA,
        ],
    )
    def onehot(idx_hbm, out_hbm, idx_v, buf_v, off_v, ones_v, sem):
        wid = lax.axis_index("s") * N_CORES + lax.axis_index("c")
        base = wid * rpw
        pltpu.sync_copy(idx_hbm.at[pl.ds(base, rpw)], idx_v)

        zeros16 = jnp.zeros((16,), jnp.float32)

        def zbody(i, carry):
            buf_v[pl.ds(i * 16, 16)] = zeros16
            return carry

        lax.fori_loop(0, chunk // 16, zbody, 0)
        pltpu.sync_copy(buf_v, out_hbm.at[pl.ds(base * n_experts, chunk)])

        iota16 = lax.broadcasted_iota(jnp.int32, (16,), 0)
        ones16 = jnp.ones((16,), jnp.float32)
        for g in range(rpw // 16):
            idx16 = idx_v[pl.ds(g * 16, 16)]
            off16 = (base + g * 16 + iota16) * n_experts + idx16
            off_v[g // 8, pl.ds((g % 8) * 16, 16)] = off16
            ones_v[pl.ds((g % 8) * 16, 16)] = ones16

        pltpu.async_copy(ones_v, out_hbm.at[off_v.at[0]], sem).wait()
        pltpu.async_copy(ones_v, out_hbm.at[off_v.at[1]], sem).wait()

    return onehot(idx_flat)


def kernel(x, W, b):
    tokens = x.shape[0]
    n_experts = W.shape[0]
    idx = _tc_argmax(x, W, b).reshape(tokens)
    out_flat = _sc_onehot(idx, tokens, n_experts)
    return out_flat.reshape(tokens, n_experts)


# speedup vs baseline: 1.7129x; 1.7129x over previous
import jax
import jax.numpy as jnp
from jax.experimental import pallas as pl
from jax.experimental.pallas import tpu as pltpu

TILE_M = 512


def _gate_kernel(x_ref, wt_ref, b_ref, out_ref):
    logits = jnp.dot(x_ref[...], wt_ref[...],
                     preferred_element_type=jnp.float32) + b_ref[...]
    m = jnp.max(logits, axis=1, keepdims=True)
    e = logits.shape[1]
    iota = jax.lax.broadcasted_iota(jnp.int32, logits.shape, 1)
    idx = jnp.min(jnp.where(logits == m, iota, e), axis=1, keepdims=True)
    out_ref[...] = (iota == idx).astype(jnp.float32)


def kernel(x, W, b):
    tokens, d_model = x.shape
    n_experts = W.shape[0]
    grid = (tokens // TILE_M,)
    return pl.pallas_call(
        _gate_kernel,
        grid=grid,
        in_specs=[
            pl.BlockSpec((TILE_M, d_model), lambda i: (i, 0)),
            pl.BlockSpec((d_model, n_experts), lambda i: (0, 0)),
            pl.BlockSpec((1, n_experts), lambda i: (0, 0)),
        ],
        out_specs=pl.BlockSpec((TILE_M, n_experts), lambda i: (i, 0)),
        out_shape=jax.ShapeDtypeStruct((tokens, n_experts), jnp.float32),
        compiler_params=pltpu.CompilerParams(
            dimension_semantics=("arbitrary",),
        ),
    )(x, W.T, b.reshape(1, n_experts))


# speedup vs baseline: 1.8303x; 1.0686x over previous
import jax
import jax.numpy as jnp
from jax.experimental import pallas as pl
from jax.experimental.pallas import tpu as pltpu

TILE_M = 512


def _gate_kernel(x_ref, w_ref, b_ref, out_ref):
    logits = jax.lax.dot_general(
        x_ref[...], w_ref[...], (((1,), (1,)), ((), ())),
        preferred_element_type=jnp.float32) + b_ref[...]
    m = jnp.max(logits, axis=1, keepdims=True)
    e = logits.shape[1]
    iota = jax.lax.broadcasted_iota(jnp.int32, logits.shape, 1)
    idx = jnp.min(jnp.where(logits == m, iota, e), axis=1, keepdims=True)
    out_ref[...] = (iota == idx).astype(jnp.float32)


def kernel(x, W, b):
    tokens, d_model = x.shape
    n_experts = W.shape[0]
    grid = (tokens // TILE_M,)
    return pl.pallas_call(
        _gate_kernel,
        grid=grid,
        in_specs=[
            pl.BlockSpec((TILE_M, d_model), lambda i: (i, 0)),
            pl.BlockSpec((n_experts, d_model), lambda i: (0, 0)),
            pl.BlockSpec((1, n_experts), lambda i: (0, 0)),
        ],
        out_specs=pl.BlockSpec((TILE_M, n_experts), lambda i: (i, 0)),
        out_shape=jax.ShapeDtypeStruct((tokens, n_experts), jnp.float32),
        compiler_params=pltpu.CompilerParams(
            dimension_semantics=("arbitrary",),
        ),
    )(x, W, b.reshape(1, n_experts))
